# baseline (speedup 1.0000x reference)
import numpy as np
import jax
import jax.numpy as jnp
from jax.experimental import pallas as pl

G = 8
T = 32
TAU_TOPK = 0.5
EPS = 1e-06
K_TOP = 4
PER_G = 256
BLK_R = 2048
SUB = 128

_KS0 = np.uint32(0)
_KS1 = np.uint32(42)
_KS2 = np.uint32(0 ^ 42 ^ 0x1BD11BDA)
_ROT_A = (13, 15, 26, 6)
_ROT_B = (17, 29, 16, 24)


def _rotl(v, d):
    return (v << np.uint32(d)) | (v >> np.uint32(32 - d))


def _rounds(x0, x1, rots):
    for r in rots:
        x0 = x0 + x1
        x1 = _rotl(x1, r) ^ x0
    return x0, x1


def _threefry_bits(x1):
    x0 = x1
    x1 = _rotl(x1, _ROT_A[0]) ^ x0
    x0, x1 = _rounds(x0, x1, _ROT_A[1:])
    x0 = x0 + _KS1
    x1 = x1 + (_KS2 + np.uint32(1))
    x0, x1 = _rounds(x0, x1, _ROT_B)
    x0 = x0 + _KS2
    x1 = x1 + (_KS0 + np.uint32(2))
    x0, x1 = _rounds(x0, x1, _ROT_A)
    x0 = x0 + _KS0
    x1 = x1 + (_KS1 + np.uint32(3))
    x0, x1 = _rounds(x0, x1, _ROT_B)
    x0 = x0 + _KS1
    x1 = x1 + (_KS2 + np.uint32(4))
    x0, x1 = _rounds(x0, x1, _ROT_A)
    x0 = x0 + _KS2
    x1 = x1 + (_KS0 + np.uint32(5))
    return x0 ^ x1


def _body(x_ref, pr_ref, up_ref, o_ref):
    pid = pl.program_id(0)

    up = up_ref[...]
    pr = pr_ref[...]

    logits = pr * np.float32(1.0 / TAU_TOPK)
    m = jnp.max(logits, axis=1, keepdims=True)
    e = jnp.exp(logits - m)
    w = e / jnp.sum(e, axis=1, keepdims=True)
    sum_w = jnp.maximum(jnp.sum(w, axis=1, keepdims=True), 1e-12)
    mask_soft = w * (np.float32(K_TOP) / sum_w)
    p = jax.nn.sigmoid(pr * mask_soft)
    p = jnp.clip(p, EPS, 1.0 - EPS)
    tint = jnp.ceil(p * np.float32(1 << 23))

    rg = jax.lax.broadcasted_iota(jnp.int32, (SUB, G), 0)
    cg = jax.lax.broadcasted_iota(jnp.int32, (SUB, G), 1)
    sel = (rg & (G - 1)) == cg
    tm = jnp.where(sel, jnp.broadcast_to(tint, (SUB, G)), np.float32(0.0))
    thr9 = jnp.sum(tm, axis=1, keepdims=True).astype(jnp.uint32) << np.uint32(9)

    tscale = np.float32(T) / up

    r_io = jax.lax.broadcasted_iota(jnp.uint32, (SUB, PER_G), 0)
    c_io = jax.lax.broadcasted_iota(jnp.uint32, (SUB, PER_G), 1)
    base = (pid * np.int32(BLK_R * PER_G)).astype(jnp.uint32)
    x1_init = ((r_io << np.uint32(8)) | c_io) + (base + _KS1)

    def tile(s, x1c):
        bits = _threefry_bits(x1c)
        xt = x_ref[pl.ds(s * SUB, SUB), :]
        z = xt * tscale + np.float32(0.5)
        y = jnp.clip(jnp.floor(z) * np.float32(1.0 / T), 0.0, 1.0) * up
        o_ref[pl.ds(s * SUB, SUB), :] = jnp.where(bits < thr9, y, np.float32(0.0))
        return x1c + np.uint32(SUB * PER_G)

    jax.lax.fori_loop(0, BLK_R // SUB, tile, x1_init)


def kernel(x, up, probs):
    B, HW, C = x.shape
    rows = B * HW * C // PER_G

    x2 = x.reshape(rows, PER_G)
    pr = probs.reshape(1, G)
    up2 = up.reshape(1, 1)

    out = pl.pallas_call(
        _body,
        grid=(rows // BLK_R,),
        in_specs=[
            pl.BlockSpec((BLK_R, PER_G), lambda i: (i, 0)),
            pl.BlockSpec((1, G), lambda i: (0, 0)),
            pl.BlockSpec((1, 1), lambda i: (0, 0)),
        ],
        out_specs=pl.BlockSpec((BLK_R, PER_G), lambda i: (i, 0)),
        out_shape=jax.ShapeDtypeStruct((rows, PER_G), jnp.float32),
    )(x2, pr, up2)
    return out.reshape(B, HW, C)


# speedup vs baseline: 1.1550x; 1.1550x over previous
import numpy as np
import jax
import jax.numpy as jnp
from jax.experimental import pallas as pl

G = 8
T = 32
TAU_TOPK = 0.5
EPS = 1e-06
K_TOP = 4
PER_G = 256
BLK_R = 2048
SUB = 32

_KS0 = np.uint32(0)
_KS1 = np.uint32(42)
_KS2 = np.uint32(0 ^ 42 ^ 0x1BD11BDA)
_ROT_A = (13, 15, 26, 6)
_ROT_B = (17, 29, 16, 24)


def _rotl(v, d):
    return (v << np.uint32(d)) | (v >> np.uint32(32 - d))


def _rounds(x0, x1, rots):
    for r in rots:
        x0 = x0 + x1
        x1 = _rotl(x1, r) ^ x0
    return x0, x1


def _threefry_bits(x1):
    x0 = x1
    x1 = _rotl(x1, _ROT_A[0]) ^ x0
    x0, x1 = _rounds(x0, x1, _ROT_A[1:])
    x0 = x0 + _KS1
    x1 = x1 + (_KS2 + np.uint32(1))
    x0, x1 = _rounds(x0, x1, _ROT_B)
    x0 = x0 + _KS2
    x1 = x1 + (_KS0 + np.uint32(2))
    x0, x1 = _rounds(x0, x1, _ROT_A)
    x0 = x0 + _KS0
    x1 = x1 + (_KS1 + np.uint32(3))
    x0, x1 = _rounds(x0, x1, _ROT_B)
    x0 = x0 + _KS1
    x1 = x1 + (_KS2 + np.uint32(4))
    x0, x1 = _rounds(x0, x1, _ROT_A)
    x0 = x0 + _KS2
    x1 = x1 + (_KS0 + np.uint32(5))
    return x0 ^ x1


def _body(x_ref, pr_ref, up_ref, o_ref):
    pid = pl.program_id(0)

    up = up_ref[...]
    pr = pr_ref[...]

    logits = pr * np.float32(1.0 / TAU_TOPK)
    m = jnp.max(logits, axis=1, keepdims=True)
    e = jnp.exp(logits - m)
    w = e / jnp.sum(e, axis=1, keepdims=True)
    sum_w = jnp.maximum(jnp.sum(w, axis=1, keepdims=True), 1e-12)
    mask_soft = w * (np.float32(K_TOP) / sum_w)
    p = jax.nn.sigmoid(pr * mask_soft)
    p = jnp.clip(p, EPS, 1.0 - EPS)
    tint = jnp.ceil(p * np.float32(1 << 23))

    rg = jax.lax.broadcasted_iota(jnp.int32, (SUB, G), 0)
    cg = jax.lax.broadcasted_iota(jnp.int32, (SUB, G), 1)
    sel = (rg & (G - 1)) == cg
    tm = jnp.where(sel, jnp.broadcast_to(tint, (SUB, G)), np.float32(0.0))
    thr9 = jnp.sum(tm, axis=1, keepdims=True).astype(jnp.uint32) << np.uint32(9)

    tscale = np.float32(T) / up

    r_io = jax.lax.broadcasted_iota(jnp.uint32, (SUB, PER_G), 0)
    c_io = jax.lax.broadcasted_iota(jnp.uint32, (SUB, PER_G), 1)
    base = (pid * np.int32(BLK_R * PER_G)).astype(jnp.uint32)
    x1_init = ((r_io << np.uint32(8)) | c_io) + (base + _KS1)

    for s in range(BLK_R // SUB):
        x1c = x1_init + np.uint32(s * SUB * PER_G)
        bits = _threefry_bits(x1c)
        xt = x_ref[pl.ds(s * SUB, SUB), :]
        z = xt * tscale + np.float32(0.5)
        y = jnp.clip(jnp.floor(z) * np.float32(1.0 / T), 0.0, 1.0) * up
        o_ref[pl.ds(s * SUB, SUB), :] = jnp.where(bits < thr9, y, np.float32(0.0))


def kernel(x, up, probs):
    B, HW, C = x.shape
    rows = B * HW * C // PER_G

    x2 = x.reshape(rows, PER_G)
    pr = probs.reshape(1, G)
    up2 = up.reshape(1, 1)

    out = pl.pallas_call(
        _body,
        grid=(rows // BLK_R,),
        in_specs=[
            pl.BlockSpec((BLK_R, PER_G), lambda i: (i, 0)),
            pl.BlockSpec((1, G), lambda i: (0, 0)),
            pl.BlockSpec((1, 1), lambda i: (0, 0)),
        ],
        out_specs=pl.BlockSpec((BLK_R, PER_G), lambda i: (i, 0)),
        out_shape=jax.ShapeDtypeStruct((rows, PER_G), jnp.float32),
    )(x2, pr, up2)
    return out.reshape(B, HW, C)
